# baseline (speedup 1.0000x reference)
import functools
import math

import jax
import jax.numpy as jnp
import numpy as np
from jax.experimental import pallas as pl
from jax.experimental.pallas import tpu as pltpu

N_NODES = 10000
N_EDGES = 160000
LMAX = 2
M = (LMAX + 1) ** 2
NBASIS = 8
MSG = 4
ND = MSG * NBASIS
NCH = 16
DIM = 128
CUTOFF = 5.0
NSPEC = 16


def _fact(n):
    return math.factorial(int(n))


def _su2_cg(j1, m1, j2, m2, j3, m3):
    if m1 + m2 != m3:
        return 0.0
    if not (abs(j1 - j2) <= j3 <= j1 + j2):
        return 0.0
    pref = math.sqrt((2 * j3 + 1) * _fact(j3 + j1 - j2) * _fact(j3 - j1 + j2) * _fact(j1 + j2 - j3) / _fact(j1 + j2 + j3 + 1))
    pref *= math.sqrt(_fact(j3 + m3) * _fact(j3 - m3) * _fact(j1 - m1) * _fact(j1 + m1) * _fact(j2 - m2) * _fact(j2 + m2))
    s = 0.0
    for k in range(0, j1 + j2 - j3 + 1):
        denoms = [k, j1 + j2 - j3 - k, j1 - m1 - k, j2 + m2 - k, j3 - j2 + m1 + k, j3 - j1 - m2 + k]
        if any(d < 0 for d in denoms):
            continue
        d = 1.0
        for x in denoms:
            d *= _fact(x)
        s += (-1) ** k / d
    return pref * s


def _Q(l):
    q = np.zeros((2 * l + 1, 2 * l + 1), dtype=np.complex128)
    for m in range(-l, 0):
        q[l + m, l + abs(m)] = 1.0 / np.sqrt(2)
        q[l + m, l - abs(m)] = -1j / np.sqrt(2)
    q[l, l] = 1.0
    for m in range(1, l + 1):
        q[l + m, l + abs(m)] = (-1) ** m / np.sqrt(2)
        q[l + m, l - abs(m)] = 1j * (-1) ** m / np.sqrt(2)
    return ((-1j) ** l) * q


def _real_cg(l1, l2, l3):
    C = np.zeros((2 * l1 + 1, 2 * l2 + 1, 2 * l3 + 1), dtype=np.complex128)
    for m1 in range(-l1, l1 + 1):
        for m2 in range(-l2, l2 + 1):
            for m3 in range(-l3, l3 + 1):
                C[l1 + m1, l2 + m2, l3 + m3] = _su2_cg(l1, m1, l2, m2, l3, m3)
    K = np.einsum('abc,ai,bj,ck->ijk', C, np.conj(_Q(l1)), np.conj(_Q(l2)), _Q(l3))
    return (K.real + K.imag).astype(np.float32)


_offs = [0, 1, 4]
_paths = [(l1, l2, l3) for l1 in range(LMAX + 1) for l2 in range(LMAX + 1) for l3 in range(LMAX + 1) if abs(l1 - l2) <= l3 <= l1 + l2]
NPATHS = len(_paths)
_CGnp = np.zeros((NPATHS, M, M, M), dtype=np.float32)
for _p, (_l1, _l2, _l3) in enumerate(_paths):
    _CGnp[_p, _offs[_l1]:_offs[_l1] + 2 * _l1 + 1, _offs[_l2]:_offs[_l2] + 2 * _l2 + 1, _offs[_l3]:_offs[_l3] + 2 * _l3 + 1] = _real_cg(_l1, _l2, _l3)
_CG = jnp.asarray(_CGnp)


def _edge_prep_body(dist_ref, vec_ref, switch_ref, rb_ref, y_ref):
    d = dist_ref[:, 0]
    inv = 1.0 / d
    x = vec_ref[:, 0] * inv
    y = vec_ref[:, 1] * inv
    z = vec_ref[:, 2] * inv
    s3 = math.sqrt(3.0)
    s15 = math.sqrt(15.0)
    s5 = math.sqrt(5.0)
    ys = [
        jnp.ones_like(x),
        s3 * x, s3 * y, s3 * z,
        s15 * x * y, s15 * y * z, 0.5 * s5 * (3.0 * z * z - 1.0), s15 * x * z, 0.5 * s15 * (x * x - y * y),
    ]
    y_ref[...] = jnp.stack(ys, axis=-1)
    sw = switch_ref[:, 0] * inv * math.sqrt(2.0 / CUTOFF)
    arg = d * (math.pi / CUTOFF)
    rbs = [jnp.sin(arg * n) * sw for n in range(1, NBASIS + 1)]
    rb_ref[...] = jnp.stack(rbs, axis=-1)


def _edge_prep(distances, vec, switch):
    E = distances.shape[0]
    BE = 2000
    grid = (E // BE,)
    return pl.pallas_call(
        _edge_prep_body,
        grid=grid,
        in_specs=[
            pl.BlockSpec((BE, 1), lambda i: (i, 0)),
            pl.BlockSpec((BE, 3), lambda i: (i, 0)),
            pl.BlockSpec((BE, 1), lambda i: (i, 0)),
        ],
        out_specs=[
            pl.BlockSpec((BE, NBASIS), lambda i: (i, 0)),
            pl.BlockSpec((BE, M), lambda i: (i, 0)),
        ],
        out_shape=[
            jax.ShapeDtypeStruct((E, NBASIS), jnp.float32),
            jax.ShapeDtypeStruct((E, M), jnp.float32),
        ],
    )(distances[:, None], vec, switch[:, None])


def _mixE3(x, W):
    Wf = jnp.concatenate([jnp.repeat(W[l][None], 2 * l + 1, axis=0) for l in range(LMAX + 1)], axis=0)
    return jnp.einsum('moi,nim->nom', Wf, x)


def _ftp(x, y, w):
    C2 = jnp.einsum('pc,pijk->cijk', w, _CG)
    return jnp.einsum('cijk,nci,ncj->nck', C2, x, y)


def kernel(species, edge_src, edge_dst, distances, vec, switch, params):
    rb, Yij = _edge_prep(distances, vec, switch)
    Yij = Yij[:, None, :]
    onehot = jnp.eye(NSPEC, dtype=jnp.float32)[species]
    xi = onehot @ params['W_species'] + params['b_species']
    N = species.shape[0]
    density = None
    Vi = None
    for layer in range(2):
        mi = xi @ params['Wsl%d' % layer] + params['bsl%d' % layer]
        xij = (mi[edge_dst][:, :, None] * rb[:, None, :]).reshape(-1, ND)
        if layer == 0:
            rhoij = xij[:, :, None] * Yij
            density = jax.ops.segment_sum(rhoij, edge_src, N)
            Vi = _mixE3(density, params['W_Vi_init'])
        else:
            rhoi = _mixE3(Vi, params['W_rho_mix_1'])
            rhoij = xij[:, :, None] * rhoi[edge_dst]
            density = density + jax.ops.segment_sum(rhoij, edge_src, N)
        scals = [density[:, :, 0]]
        for i in range(2):
            Hi = jnp.einsum('oi,nim->nom', params['W_dm_%d_%d' % (layer, i)], density)
            Li = _ftp(Vi, Hi, params['W_tp_%d_%d' % (layer, i)])
            scals.append(Li[:, :, 0])
            Vi = Vi + Li
        h = jnp.concatenate([xi] + scals, axis=-1)
        h = jax.nn.silu(h @ params['W_l%d_0' % layer] + params['b_l%d_0' % layer])
        xi = xi + h @ params['W_l%d_1' % layer] + params['b_l%d_1' % layer]
    return xi, Vi
